# baseline (speedup 1.0000x reference)
import jax
import jax.numpy as jnp
from jax import lax
from jax.experimental import pallas as pl
from jax.experimental.pallas import tpu as pltpu
from jax.experimental.pallas import tpu_sc as plsc

BATCH = 16384
NF = 16
NC, NS = 2, 16
NW = NC * NS
BPW = BATCH // NW
L = 16
CHUNKS = BPW // L


def _sc_body(p_hbm, poll_hbm, pe_hbm, qe_hbm, pb_hbm, qb_hbm, out_hbm,
             idx_p, idx_q, pe_v, qe_v, pb_v, qb_v, res_v, sem):
    wid = lax.axis_index("s") * NC + lax.axis_index("c")
    base = wid * BPW

    pltpu.sync_copy(p_hbm.at[pl.ds(base, BPW)], idx_p)
    pltpu.sync_copy(poll_hbm.at[pl.ds(base, BPW)], idx_q)

    c1 = pltpu.async_copy(pe_hbm.at[idx_p], pe_v, sem)
    c2 = pltpu.async_copy(qe_hbm.at[idx_q], qe_v, sem)
    c3 = pltpu.async_copy(pb_hbm.at[idx_p], pb_v, sem)
    c4 = pltpu.async_copy(qb_hbm.at[idx_q], qb_v, sem)
    c1.wait()
    c2.wait()
    c3.wait()
    c4.wait()

    lanes = lax.iota(jnp.int32, L)
    zeros = jnp.zeros((L,), jnp.int32)

    def chunk(i, carry):
        rows = i * L + lanes
        acc = jnp.zeros((L,), jnp.float32)
        for f in range(NF):
            col = jnp.full((L,), f, jnp.int32)
            a = plsc.load_gather(pe_v, [rows, col])
            b = plsc.load_gather(qe_v, [rows, col])
            acc = acc + a * b
        x = acc + plsc.load_gather(pb_v, [rows, zeros]) \
                + plsc.load_gather(qb_v, [rows, zeros])
        res_v[pl.ds(i * L, L)] = 1.0 / (1.0 + jnp.exp(-x))
        return carry

    lax.fori_loop(0, CHUNKS, chunk, 0)

    pltpu.sync_copy(res_v, out_hbm.at[pl.ds(base, BPW)])


@jax.jit
def _run(p, poll, p_embed, poll_embed, p_bias, poll_bias):
    mesh = plsc.VectorSubcoreMesh(core_axis_name="c", subcore_axis_name="s")
    f = pl.kernel(
        _sc_body,
        out_type=jax.ShapeDtypeStruct((BATCH,), jnp.float32),
        mesh=mesh,
        compiler_params=pltpu.CompilerParams(needs_layout_passes=False,
                                              use_tc_tiling_on_sc=False),
        scratch_types=[
            pltpu.VMEM((BPW,), jnp.int32),
            pltpu.VMEM((BPW,), jnp.int32),
            pltpu.VMEM((BPW, NF), jnp.float32),
            pltpu.VMEM((BPW, NF), jnp.float32),
            pltpu.VMEM((BPW, 1), jnp.float32),
            pltpu.VMEM((BPW, 1), jnp.float32),
            pltpu.VMEM((BPW,), jnp.float32),
            pltpu.SemaphoreType.DMA,
        ],
    )
    return f(p, poll, p_embed, poll_embed, p_bias, poll_bias)


def kernel(p, poll, p_embed, poll_embed, p_bias, poll_bias):
    return _run(p.astype(jnp.int32), poll.astype(jnp.int32),
                p_embed, poll_embed, p_bias, poll_bias)


# speedup vs baseline: 2.6065x; 2.6065x over previous
import jax
import jax.numpy as jnp
from jax import lax
from jax.experimental import pallas as pl
from jax.experimental.pallas import tpu as pltpu
from jax.experimental.pallas import tpu_sc as plsc

BATCH = 16384
NF = 16
NC, NS = 2, 16
NW = NC * NS
BPW = BATCH // NW
L = 16
CHUNKS = BPW // L


def _sc_body(p_hbm, poll_hbm, pe_hbm, qe_hbm, pb_hbm, qb_hbm, out_hbm,
             idx_p, idx_q, pe_v, qe_v, pb_v, qb_v, res_v, sem):
    wid = lax.axis_index("s") * NC + lax.axis_index("c")
    base = wid * BPW

    pltpu.sync_copy(p_hbm.at[pl.ds(base, BPW)], idx_p)
    pltpu.sync_copy(poll_hbm.at[pl.ds(base, BPW)], idx_q)

    c1 = pltpu.async_copy(pe_hbm.at[idx_p], pe_v, sem)
    c2 = pltpu.async_copy(qe_hbm.at[idx_q], qe_v, sem)
    c3 = pltpu.async_copy(pb_hbm.at[idx_p], pb_v, sem)
    c4 = pltpu.async_copy(qb_hbm.at[idx_q], qb_v, sem)
    c1.wait()
    c2.wait()
    c3.wait()
    c4.wait()

    lanes = lax.iota(jnp.int32, L)

    def chunk(i, carry):
        rows = i * L + lanes
        acc = jnp.zeros((L,), jnp.float32)
        for f in range(NF):
            col = jnp.full((L,), f, jnp.int32)
            a = plsc.load_gather(pe_v, [rows, col])
            b = plsc.load_gather(qe_v, [rows, col])
            acc = acc + a * b
        x = acc + pb_v[pl.ds(i * L, L)] + qb_v[pl.ds(i * L, L)]
        res_v[pl.ds(i * L, L)] = 1.0 / (1.0 + jnp.exp(-x))
        return carry

    lax.fori_loop(0, CHUNKS, chunk, 0)

    pltpu.sync_copy(res_v, out_hbm.at[pl.ds(base, BPW)])


@jax.jit
def _run(p, poll, p_embed, poll_embed, p_bias, poll_bias):
    mesh = plsc.VectorSubcoreMesh(core_axis_name="c", subcore_axis_name="s")
    f = pl.kernel(
        _sc_body,
        out_type=jax.ShapeDtypeStruct((BATCH,), jnp.float32),
        mesh=mesh,
        compiler_params=pltpu.CompilerParams(needs_layout_passes=False,
                                              use_tc_tiling_on_sc=False),
        scratch_types=[
            pltpu.VMEM((BPW,), jnp.int32),
            pltpu.VMEM((BPW,), jnp.int32),
            pltpu.VMEM((BPW, NF), jnp.float32),
            pltpu.VMEM((BPW, NF), jnp.float32),
            pltpu.VMEM((BPW,), jnp.float32),
            pltpu.VMEM((BPW,), jnp.float32),
            pltpu.VMEM((BPW,), jnp.float32),
            pltpu.SemaphoreType.DMA,
        ],
    )
    return f(p, poll, p_embed, poll_embed, p_bias, poll_bias)


def kernel(p, poll, p_embed, poll_embed, p_bias, poll_bias):
    return _run(p.astype(jnp.int32), poll.astype(jnp.int32),
                p_embed, poll_embed,
                p_bias.reshape(-1), poll_bias.reshape(-1))
